# baseline (speedup 1.0000x reference)
import functools

import jax
import jax.numpy as jnp
from jax import lax
from jax.experimental import pallas as pl
from jax.experimental.pallas import tpu as pltpu
from jax.experimental.pallas import tpu_sc as plsc

_RADIUS = 0.1
_NSAMPLE = 32
_BIG = 1e10


def _select_body(q_ref, st_ref, smask_ref, qmask_ref, idx_ref, val_ref,
                 gxyz_ref):
    b = pl.program_id(0)
    ns = st_ref.shape[2]
    tq = q_ref.shape[1]
    k = _NSAMPLE
    r2 = _RADIUS * _RADIUS

    q = q_ref[0]
    st = st_ref[0]
    qx, qy, qz = q[:, 0:1], q[:, 1:2], q[:, 2:3]
    sx, sy, sz = st[0:1, :], st[1:2, :], st[2:3, :]

    q2 = qx * qx + qy * qy + qz * qz
    s2 = sx * sx + sy * sy + sz * sz
    dot = lax.dot_general(q, st, (((1,), (0,)), ((), ())),
                          preferred_element_type=jnp.float32)
    d2 = q2 + s2 - 2.0 * dot

    smask = smask_ref[0]
    d2m = jnp.where(smask > 0.0, d2, _BIG)
    lane = lax.broadcasted_iota(jnp.int32, (tq, ns), 1)

    gmin = jnp.min(d2m, axis=1, keepdims=True)
    gidx = jnp.min(jnp.where(d2m == gmin, lane, ns), axis=1, keepdims=True)

    vals0 = jnp.where(d2m <= r2, d2m, _BIG)
    slot = lax.broadcasted_iota(jnp.int32, (tq, k), 1)
    nch = ns // 128
    lane128 = lax.broadcasted_iota(jnp.int32, (tq, 128), 1)

    vals = vals0
    pv_rows, pg_rows = [], []
    for _r in range(5):
        m = vals[:, 0:128]
        ac = jnp.zeros((tq, 128), jnp.int32)
        for c in range(1, nch):
            vc = vals[:, c * 128:(c + 1) * 128]
            lt = vc < m
            m = jnp.where(lt, vc, m)
            ac = jnp.where(lt, c, ac)
        chunks = []
        for c in range(nch):
            vc = vals[:, c * 128:(c + 1) * 128]
            chunks.append(jnp.where(ac == c, _BIG, vc))
        vals = jnp.concatenate(chunks, axis=1)
        pv_rows.append(m)
        pg_rows.append(ac * 128 + lane128)
    leftover = jnp.min(vals, axis=1, keepdims=True)

    nr = len(pv_rows)
    big_i = ns * nch

    def ext_body(j, carry):
        vr = list(carry[:nr])
        ir = list(carry[nr:2 * nr])
        oidx, oval = carry[2 * nr:]
        mn = jnp.min(vr[0], axis=1, keepdims=True)
        am = jnp.min(jnp.where(vr[0] == mn, ir[0], big_i), axis=1,
                     keepdims=True)
        oval = jnp.where(slot == j, mn, oval)
        oidx = jnp.where(slot == j, am, oidx)
        hit = ir[0] == am
        for r in range(nr - 1):
            vr[r] = jnp.where(hit, vr[r + 1], vr[r])
            ir[r] = jnp.where(hit, ir[r + 1], ir[r])
        vr[nr - 1] = jnp.where(hit, _BIG, vr[nr - 1])
        ir[nr - 1] = jnp.where(hit, big_i, ir[nr - 1])
        return (*vr, *ir, oidx, oval)

    oidx = jnp.zeros((tq, k), jnp.int32)
    oval = jnp.full((tq, k), _BIG, jnp.float32)
    carry = lax.fori_loop(0, k, ext_body,
                          (*pv_rows, *pg_rows, oidx, oval))
    oidx, oval = carry[2 * nr], carry[2 * nr + 1]

    bad = jnp.any((leftover <= r2) & (leftover <= oval[:, k - 1:k]))

    def naive_fn(_):
        def body(j, carry):
            va, oi, ov = carry
            mn = jnp.min(va, axis=1, keepdims=True)
            am = jnp.min(jnp.where(va == mn, lane, ns), axis=1,
                         keepdims=True)
            ov = jnp.where(slot == j, mn, ov)
            oi = jnp.where(slot == j, am, oi)
            va = jnp.where(lane == am, _BIG, va)
            return va, oi, ov

        oi = jnp.zeros((tq, k), jnp.int32)
        ov = jnp.full((tq, k), _BIG, jnp.float32)
        _, oi, ov = lax.fori_loop(0, k, body, (vals0, oi, ov))
        return oi, ov

    oidx, oval = lax.cond(bad, naive_fn, lambda _: (oidx, oval), 0)

    qmask = qmask_ref[0]
    valid = (oval <= r2) & (qmask[0][:, None] > 0.0)
    idx = jnp.where(valid, oidx, gidx)
    idx_ref[0] = idx + b * ns
    val_ref[0] = valid.astype(jnp.float32)

    ic = idx // 128
    il = idx - ic * 128
    selx = jnp.zeros((tq, k), jnp.float32)
    sely = jnp.zeros((tq, k), jnp.float32)
    selz = jnp.zeros((tq, k), jnp.float32)
    for cidx in range(ns // 128):
        m = ic == cidx
        cs = slice(cidx * 128, (cidx + 1) * 128)
        gxc = jnp.take_along_axis(jnp.broadcast_to(sx[:, cs], (tq, 128)), il, axis=1)
        gyc = jnp.take_along_axis(jnp.broadcast_to(sy[:, cs], (tq, 128)), il, axis=1)
        gzc = jnp.take_along_axis(jnp.broadcast_to(sz[:, cs], (tq, 128)), il, axis=1)
        selx = jnp.where(m, gxc, selx)
        sely = jnp.where(m, gyc, sely)
        selz = jnp.where(m, gzc, selz)
    gx = [(selx - qx)[None], (sely - qy)[None], (selz - qz)[None]]
    gxyz_ref[0] = jnp.concatenate(gx, axis=0)


def _ball_query(query_xyz, support_t, smaskf, qmaskf, tq=64):
    B, nq, _ = query_xyz.shape
    ns = support_t.shape[2]
    nt = nq // tq
    grid = (B, nt)
    return pl.pallas_call(
        _select_body,
        grid=grid,
        in_specs=[
            pl.BlockSpec((1, tq, 3), lambda b, t: (b, t, 0)),
            pl.BlockSpec((1, 3, ns), lambda b, t: (b, 0, 0)),
            pl.BlockSpec((1, 1, ns), lambda b, t: (b, 0, 0)),
            pl.BlockSpec((1, 1, tq), lambda b, t: (b * nt + t, 0, 0)),
        ],
        out_specs=[
            pl.BlockSpec((1, tq, _NSAMPLE), lambda b, t: (b, t, 0)),
            pl.BlockSpec((1, tq, _NSAMPLE), lambda b, t: (b, t, 0)),
            pl.BlockSpec((1, 3, tq, _NSAMPLE), lambda b, t: (b, 0, t, 0)),
        ],
        out_shape=[
            jax.ShapeDtypeStruct((B, nq, _NSAMPLE), jnp.int32),
            jax.ShapeDtypeStruct((B, nq, _NSAMPLE), jnp.float32),
            jax.ShapeDtypeStruct((B, 3, nq, _NSAMPLE), jnp.float32),
        ],
        compiler_params=pltpu.CompilerParams(
            dimension_semantics=("parallel", "parallel")),
    )(query_xyz, support_t, smaskf, qmaskf)


def _sc_gather(feat2d, idx_flat):
    n, d = idx_flat.shape[0], feat2d.shape[1]
    info = plsc.get_sparse_core_info()
    nc = info.num_cores
    nw = nc * info.num_subcores
    per_w = n // nw
    ch = 256
    n_iter = per_w // ch
    mesh = plsc.VectorSubcoreMesh(core_axis_name="c", subcore_axis_name="s")

    @functools.partial(
        pl.kernel,
        mesh=mesh,
        out_type=jax.ShapeDtypeStruct((n, d), jnp.float32),
        scratch_types=[
            pltpu.VMEM((ch,), jnp.int32),
            pltpu.VMEM((ch, d), jnp.float32),
            pltpu.SemaphoreType.DMA,
        ],
    )
    def kk(feat_hbm, idx_hbm, outf_hbm, idx_v, rows_v, sem):
        wid = lax.axis_index("s") * nc + lax.axis_index("c")
        base = wid * per_w

        def body(i, _):
            off = pl.multiple_of(base + i * ch, ch)
            pltpu.sync_copy(idx_hbm.at[pl.ds(off, ch)], idx_v)
            pltpu.async_copy(feat_hbm.at[idx_v], rows_v, sem).wait()
            pltpu.sync_copy(rows_v, outf_hbm.at[pl.ds(off, ch)])
            return 0

        lax.fori_loop(0, n_iter, body, 0)

    return kk(feat2d, idx_flat)


def _assemble_body(g_ref, x_ref, out_ref):
    tq = x_ref.shape[2]
    k = _NSAMPLE
    c = g_ref.shape[2]
    g = g_ref[0]
    x = x_ref[0]
    feat = jnp.swapaxes(g, 0, 1)
    feat = feat.reshape(c, tq, k)
    out_ref[0] = jnp.concatenate([x, feat], axis=0)


def _assemble(gfeat, gxyz, query_xyz, tq=64):
    B, nq, _ = query_xyz.shape
    k = _NSAMPLE
    c = gfeat.shape[1]
    g3 = gfeat.reshape(B, nq * k, c)
    nt = nq // tq
    return pl.pallas_call(
        _assemble_body,
        grid=(B, nt),
        in_specs=[
            pl.BlockSpec((1, tq * k, c), lambda b, t: (b, t, 0)),
            pl.BlockSpec((1, 3, tq, k), lambda b, t: (b, 0, t, 0)),
        ],
        out_specs=pl.BlockSpec((1, 3 + c, tq, k), lambda b, t: (b, 0, t, 0)),
        out_shape=jax.ShapeDtypeStruct((B, 3 + c, nq, k), jnp.float32),
        compiler_params=pltpu.CompilerParams(
            dimension_semantics=("parallel", "parallel")),
    )(g3, gxyz)


def kernel(query_xyz, support_xyz, query_mask, support_mask, features):
    B, nq, _ = query_xyz.shape
    ns = support_xyz.shape[1]
    C = features.shape[1]
    k = _NSAMPLE

    support_t = jnp.swapaxes(support_xyz, 1, 2)
    smaskf = support_mask.astype(jnp.float32).reshape(B, 1, ns)
    qmaskf = query_mask.astype(jnp.float32).reshape(B * (nq // 64), 1, 64)

    idx, validf, gxyz = _ball_query(query_xyz, support_t, smaskf, qmaskf,
                                    tq=64)

    feat2d = jnp.swapaxes(features, 1, 2).reshape(B * ns, C)
    gfeat = _sc_gather(feat2d, idx.reshape(B * nq * k))
    new_features = _assemble(gfeat, gxyz, query_xyz, tq=64)
    idx_mask = validf > 0.5
    return (new_features, idx_mask)


# speedup vs baseline: 1.1725x; 1.1725x over previous
import functools

import jax
import jax.numpy as jnp
from jax import lax
from jax.experimental import pallas as pl
from jax.experimental.pallas import tpu as pltpu
from jax.experimental.pallas import tpu_sc as plsc

_RADIUS = 0.1
_NSAMPLE = 32
_BIG = 1e10


def _select_body(q_ref, st_ref, smask_ref, qmask_ref, idx_ref, val_ref,
                 gxyz_ref):
    b = pl.program_id(0)
    ns = st_ref.shape[2]
    tq = q_ref.shape[1]
    k = _NSAMPLE
    r2 = _RADIUS * _RADIUS

    q = q_ref[0]
    st = st_ref[0]
    qx, qy, qz = q[:, 0:1], q[:, 1:2], q[:, 2:3]
    sx, sy, sz = st[0:1, :], st[1:2, :], st[2:3, :]

    q2 = qx * qx + qy * qy + qz * qz
    s2 = sx * sx + sy * sy + sz * sz
    dot = lax.dot_general(q, st, (((1,), (0,)), ((), ())),
                          preferred_element_type=jnp.float32)
    d2 = q2 + s2 - 2.0 * dot

    smask = smask_ref[0]
    d2m = jnp.where(smask > 0.0, d2, _BIG)
    lane = lax.broadcasted_iota(jnp.int32, (tq, ns), 1)

    gmin = jnp.min(d2m, axis=1, keepdims=True)
    gidx = jnp.min(jnp.where(d2m == gmin, lane, ns), axis=1, keepdims=True)

    vals0 = jnp.where(d2m <= r2, d2m, _BIG)
    slot = lax.broadcasted_iota(jnp.int32, (tq, k), 1)
    nch = ns // 128
    lane128 = lax.broadcasted_iota(jnp.int32, (tq, 128), 1)

    vals = vals0
    pv_rows, pg_rows = [], []
    for _r in range(5):
        m = vals[:, 0:128]
        ac = jnp.zeros((tq, 128), jnp.int32)
        for c in range(1, nch):
            vc = vals[:, c * 128:(c + 1) * 128]
            lt = vc < m
            m = jnp.where(lt, vc, m)
            ac = jnp.where(lt, c, ac)
        chunks = []
        for c in range(nch):
            vc = vals[:, c * 128:(c + 1) * 128]
            chunks.append(jnp.where(ac == c, _BIG, vc))
        vals = jnp.concatenate(chunks, axis=1)
        pv_rows.append(m)
        pg_rows.append(ac * 128 + lane128)
    leftover = jnp.min(vals, axis=1, keepdims=True)

    nr = len(pv_rows)
    big_i = ns * nch

    def ext_body(j, carry):
        vr = list(carry[:nr])
        ir = list(carry[nr:2 * nr])
        oidx, oval = carry[2 * nr:]
        mn = jnp.min(vr[0], axis=1, keepdims=True)
        am = jnp.min(jnp.where(vr[0] == mn, ir[0], big_i), axis=1,
                     keepdims=True)
        oval = jnp.where(slot == j, mn, oval)
        oidx = jnp.where(slot == j, am, oidx)
        hit = ir[0] == am
        for r in range(nr - 1):
            vr[r] = jnp.where(hit, vr[r + 1], vr[r])
            ir[r] = jnp.where(hit, ir[r + 1], ir[r])
        vr[nr - 1] = jnp.where(hit, _BIG, vr[nr - 1])
        ir[nr - 1] = jnp.where(hit, big_i, ir[nr - 1])
        return (*vr, *ir, oidx, oval)

    oidx = jnp.zeros((tq, k), jnp.int32)
    oval = jnp.full((tq, k), _BIG, jnp.float32)
    carry = lax.fori_loop(0, k, ext_body,
                          (*pv_rows, *pg_rows, oidx, oval))
    oidx, oval = carry[2 * nr], carry[2 * nr + 1]

    bad = jnp.any((leftover <= r2) & (leftover <= oval[:, k - 1:k]))

    def naive_fn(_):
        def body(j, carry):
            va, oi, ov = carry
            mn = jnp.min(va, axis=1, keepdims=True)
            am = jnp.min(jnp.where(va == mn, lane, ns), axis=1,
                         keepdims=True)
            ov = jnp.where(slot == j, mn, ov)
            oi = jnp.where(slot == j, am, oi)
            va = jnp.where(lane == am, _BIG, va)
            return va, oi, ov

        oi = jnp.zeros((tq, k), jnp.int32)
        ov = jnp.full((tq, k), _BIG, jnp.float32)
        _, oi, ov = lax.fori_loop(0, k, body, (vals0, oi, ov))
        return oi, ov

    oidx, oval = lax.cond(bad, naive_fn, lambda _: (oidx, oval), 0)

    qmask = qmask_ref[0]
    valid = (oval <= r2) & (qmask[0][:, None] > 0.0)
    idx = jnp.where(valid, oidx, gidx)
    idx_ref[0] = idx + b * ns
    val_ref[0] = valid.astype(jnp.float32)

    ic = idx // 128
    il = idx - ic * 128
    selx = jnp.zeros((tq, k), jnp.float32)
    sely = jnp.zeros((tq, k), jnp.float32)
    selz = jnp.zeros((tq, k), jnp.float32)
    for cidx in range(ns // 128):
        m = ic == cidx
        cs = slice(cidx * 128, (cidx + 1) * 128)
        gxc = jnp.take_along_axis(jnp.broadcast_to(sx[:, cs], (tq, 128)), il, axis=1)
        gyc = jnp.take_along_axis(jnp.broadcast_to(sy[:, cs], (tq, 128)), il, axis=1)
        gzc = jnp.take_along_axis(jnp.broadcast_to(sz[:, cs], (tq, 128)), il, axis=1)
        selx = jnp.where(m, gxc, selx)
        sely = jnp.where(m, gyc, sely)
        selz = jnp.where(m, gzc, selz)
    gx = [(selx - qx)[None], (sely - qy)[None], (selz - qz)[None]]
    gxyz_ref[0] = jnp.concatenate(gx, axis=0)


def _ball_query(query_xyz, support_t, smaskf, qmaskf, tq=64):
    B, nq, _ = query_xyz.shape
    ns = support_t.shape[2]
    nt = nq // tq
    grid = (B, nt)
    return pl.pallas_call(
        _select_body,
        grid=grid,
        in_specs=[
            pl.BlockSpec((1, tq, 3), lambda b, t: (b, t, 0)),
            pl.BlockSpec((1, 3, ns), lambda b, t: (b, 0, 0)),
            pl.BlockSpec((1, 1, ns), lambda b, t: (b, 0, 0)),
            pl.BlockSpec((1, 1, tq), lambda b, t: (b * nt + t, 0, 0)),
        ],
        out_specs=[
            pl.BlockSpec((1, tq, _NSAMPLE), lambda b, t: (b, t, 0)),
            pl.BlockSpec((1, tq, _NSAMPLE), lambda b, t: (b, t, 0)),
            pl.BlockSpec((1, 3, tq, _NSAMPLE), lambda b, t: (b, 0, t, 0)),
        ],
        out_shape=[
            jax.ShapeDtypeStruct((B, nq, _NSAMPLE), jnp.int32),
            jax.ShapeDtypeStruct((B, nq, _NSAMPLE), jnp.float32),
            jax.ShapeDtypeStruct((B, 3, nq, _NSAMPLE), jnp.float32),
        ],
        compiler_params=pltpu.CompilerParams(
            dimension_semantics=("parallel", "parallel")),
    )(query_xyz, support_t, smaskf, qmaskf)


def _sc_gather(feat2d, idx_flat):
    n, d = idx_flat.shape[0], feat2d.shape[1]
    info = plsc.get_sparse_core_info()
    nc = info.num_cores
    nw = nc * info.num_subcores
    per_w = n // nw
    ch = 256
    n_iter = per_w // ch
    mesh = plsc.VectorSubcoreMesh(core_axis_name="c", subcore_axis_name="s")

    @functools.partial(
        pl.kernel,
        mesh=mesh,
        out_type=jax.ShapeDtypeStruct((n, d), jnp.float32),
        scratch_types=[
            pltpu.VMEM((ch,), jnp.int32),
            pltpu.VMEM((ch, d), jnp.float32),
            pltpu.SemaphoreType.DMA,
        ],
    )
    def kk(feat_hbm, idx_hbm, outf_hbm, idx_v, rows_v, sem):
        wid = lax.axis_index("s") * nc + lax.axis_index("c")
        base = wid * per_w

        def body(i, _):
            off = pl.multiple_of(base + i * ch, ch)
            pltpu.sync_copy(idx_hbm.at[pl.ds(off, ch)], idx_v)
            pltpu.async_copy(feat_hbm.at[idx_v], rows_v, sem).wait()
            pltpu.sync_copy(rows_v, outf_hbm.at[pl.ds(off, ch)])
            return 0

        lax.fori_loop(0, n_iter, body, 0)

    return kk(feat2d, idx_flat)


def _assemble_body(g_ref, x_ref, out_ref):
    tq = x_ref.shape[2]
    k = _NSAMPLE
    c = g_ref.shape[2]
    g = g_ref[0]
    x = x_ref[0]
    feat = jnp.swapaxes(g, 0, 1)
    feat = feat.reshape(c, tq, k)
    out_ref[0] = jnp.concatenate([x, feat], axis=0)


def _assemble(gfeat, gxyz, query_xyz, tq=64):
    B, nq, _ = query_xyz.shape
    k = _NSAMPLE
    c = gfeat.shape[1]
    g3 = gfeat.reshape(B, nq * k, c)
    nt = nq // tq
    return pl.pallas_call(
        _assemble_body,
        grid=(B, nt),
        in_specs=[
            pl.BlockSpec((1, tq * k, c), lambda b, t: (b, t, 0)),
            pl.BlockSpec((1, 3, tq, k), lambda b, t: (b, 0, t, 0)),
        ],
        out_specs=pl.BlockSpec((1, 3 + c, tq, k), lambda b, t: (b, 0, t, 0)),
        out_shape=jax.ShapeDtypeStruct((B, 3 + c, nq, k), jnp.float32),
        compiler_params=pltpu.CompilerParams(
            dimension_semantics=("parallel", "parallel")),
    )(g3, gxyz)


def kernel(query_xyz, support_xyz, query_mask, support_mask, features):
    B, nq, _ = query_xyz.shape
    ns = support_xyz.shape[1]
    C = features.shape[1]
    k = _NSAMPLE

    tq = 128
    support_t = jnp.swapaxes(support_xyz, 1, 2)
    smaskf = support_mask.astype(jnp.float32).reshape(B, 1, ns)
    qmaskf = query_mask.astype(jnp.float32).reshape(B * (nq // tq), 1, tq)

    idx, validf, gxyz = _ball_query(query_xyz, support_t, smaskf, qmaskf,
                                    tq=tq)

    feat2d = jnp.swapaxes(features, 1, 2).reshape(B * ns, C)
    gfeat = _sc_gather(feat2d, idx.reshape(B * nq * k))
    new_features = _assemble(gfeat, gxyz, query_xyz, tq=64)
    idx_mask = validf > 0.5
    return (new_features, idx_mask)


# speedup vs baseline: 1.2676x; 1.0811x over previous
import functools

import jax
import jax.numpy as jnp
from jax import lax
from jax.experimental import pallas as pl
from jax.experimental.pallas import tpu as pltpu
from jax.experimental.pallas import tpu_sc as plsc

_RADIUS = 0.1
_NSAMPLE = 32
_BIG = 1e10


def _select_body(q_ref, st_ref, smask_ref, qmask_ref, idx_ref, val_ref,
                 gxyz_ref):
    b = pl.program_id(0)
    ns = st_ref.shape[2]
    tq = q_ref.shape[1]
    k = _NSAMPLE
    r2 = _RADIUS * _RADIUS

    q = q_ref[0]
    st = st_ref[0]
    qx, qy, qz = q[:, 0:1], q[:, 1:2], q[:, 2:3]
    sx, sy, sz = st[0:1, :], st[1:2, :], st[2:3, :]

    q2 = qx * qx + qy * qy + qz * qz
    s2 = sx * sx + sy * sy + sz * sz
    dot = lax.dot_general(q, st, (((1,), (0,)), ((), ())),
                          preferred_element_type=jnp.float32)
    d2 = q2 + s2 - 2.0 * dot

    smask = smask_ref[0]
    d2m = jnp.where(smask > 0.0, d2, _BIG)
    lane = lax.broadcasted_iota(jnp.int32, (tq, ns), 1)

    gmin = jnp.min(d2m, axis=1, keepdims=True)
    gidx = jnp.min(jnp.where(d2m == gmin, lane, ns), axis=1, keepdims=True)

    vals0 = jnp.where(d2m <= r2, d2m, _BIG)
    slot = lax.broadcasted_iota(jnp.int32, (tq, k), 1)
    nch = ns // 128
    lane128 = lax.broadcasted_iota(jnp.int32, (tq, 128), 1)

    vals = vals0
    pv_rows, pg_rows = [], []
    for _r in range(5):
        m = vals[:, 0:128]
        ac = jnp.zeros((tq, 128), jnp.int32)
        for c in range(1, nch):
            vc = vals[:, c * 128:(c + 1) * 128]
            lt = vc < m
            m = jnp.where(lt, vc, m)
            ac = jnp.where(lt, c, ac)
        chunks = []
        for c in range(nch):
            vc = vals[:, c * 128:(c + 1) * 128]
            chunks.append(jnp.where(ac == c, _BIG, vc))
        vals = jnp.concatenate(chunks, axis=1)
        pv_rows.append(m)
        pg_rows.append(ac * 128 + lane128)
    leftover = jnp.min(vals, axis=1, keepdims=True)

    nr = len(pv_rows)
    big_i = ns * nch

    def ext_body(j, carry):
        vr = list(carry[:nr])
        ir = list(carry[nr:2 * nr])
        oidx, oval = carry[2 * nr:]
        mn = jnp.min(vr[0], axis=1, keepdims=True)
        am = jnp.min(jnp.where(vr[0] == mn, ir[0], big_i), axis=1,
                     keepdims=True)
        oval = jnp.where(slot == j, mn, oval)
        oidx = jnp.where(slot == j, am, oidx)
        hit = ir[0] == am
        for r in range(nr - 1):
            vr[r] = jnp.where(hit, vr[r + 1], vr[r])
            ir[r] = jnp.where(hit, ir[r + 1], ir[r])
        vr[nr - 1] = jnp.where(hit, _BIG, vr[nr - 1])
        ir[nr - 1] = jnp.where(hit, big_i, ir[nr - 1])
        return (*vr, *ir, oidx, oval)

    oidx = jnp.zeros((tq, k), jnp.int32)
    oval = jnp.full((tq, k), _BIG, jnp.float32)
    carry = lax.fori_loop(0, k, ext_body,
                          (*pv_rows, *pg_rows, oidx, oval))
    oidx, oval = carry[2 * nr], carry[2 * nr + 1]

    bad = jnp.any((leftover <= r2) & (leftover <= oval[:, k - 1:k]))

    def naive_fn(_):
        def body(j, carry):
            va, oi, ov = carry
            mn = jnp.min(va, axis=1, keepdims=True)
            am = jnp.min(jnp.where(va == mn, lane, ns), axis=1,
                         keepdims=True)
            ov = jnp.where(slot == j, mn, ov)
            oi = jnp.where(slot == j, am, oi)
            va = jnp.where(lane == am, _BIG, va)
            return va, oi, ov

        oi = jnp.zeros((tq, k), jnp.int32)
        ov = jnp.full((tq, k), _BIG, jnp.float32)
        _, oi, ov = lax.fori_loop(0, k, body, (vals0, oi, ov))
        return oi, ov

    oidx, oval = lax.cond(bad, naive_fn, lambda _: (oidx, oval), 0)

    qmask = qmask_ref[0]
    valid = (oval <= r2) & (qmask[0][:, None] > 0.0)
    idx = jnp.where(valid, oidx, gidx)
    idx_ref[0] = idx + b * ns
    val_ref[0] = valid.astype(jnp.float32)

    ic = idx // 128
    il = idx - ic * 128
    selx = jnp.zeros((tq, k), jnp.float32)
    sely = jnp.zeros((tq, k), jnp.float32)
    selz = jnp.zeros((tq, k), jnp.float32)
    for cidx in range(ns // 128):
        m = ic == cidx
        cs = slice(cidx * 128, (cidx + 1) * 128)
        gxc = jnp.take_along_axis(jnp.broadcast_to(sx[:, cs], (tq, 128)), il, axis=1)
        gyc = jnp.take_along_axis(jnp.broadcast_to(sy[:, cs], (tq, 128)), il, axis=1)
        gzc = jnp.take_along_axis(jnp.broadcast_to(sz[:, cs], (tq, 128)), il, axis=1)
        selx = jnp.where(m, gxc, selx)
        sely = jnp.where(m, gyc, sely)
        selz = jnp.where(m, gzc, selz)
    gx = [(selx - qx)[None], (sely - qy)[None], (selz - qz)[None]]
    gxyz_ref[0] = jnp.concatenate(gx, axis=0)


def _ball_query(query_xyz, support_t, smaskf, qmaskf, tq=64):
    B, nq, _ = query_xyz.shape
    ns = support_t.shape[2]
    nt = nq // tq
    grid = (B, nt)
    return pl.pallas_call(
        _select_body,
        grid=grid,
        in_specs=[
            pl.BlockSpec((1, tq, 3), lambda b, t: (b, t, 0)),
            pl.BlockSpec((1, 3, ns), lambda b, t: (b, 0, 0)),
            pl.BlockSpec((1, 1, ns), lambda b, t: (b, 0, 0)),
            pl.BlockSpec((1, 1, tq), lambda b, t: (b * nt + t, 0, 0)),
        ],
        out_specs=[
            pl.BlockSpec((1, tq, _NSAMPLE), lambda b, t: (b, t, 0)),
            pl.BlockSpec((1, tq, _NSAMPLE), lambda b, t: (b, t, 0)),
            pl.BlockSpec((1, 3, tq, _NSAMPLE), lambda b, t: (b, 0, t, 0)),
        ],
        out_shape=[
            jax.ShapeDtypeStruct((B, nq, _NSAMPLE), jnp.int32),
            jax.ShapeDtypeStruct((B, nq, _NSAMPLE), jnp.float32),
            jax.ShapeDtypeStruct((B, 3, nq, _NSAMPLE), jnp.float32),
        ],
        compiler_params=pltpu.CompilerParams(
            dimension_semantics=("parallel", "parallel")),
    )(query_xyz, support_t, smaskf, qmaskf)


def _sc_gather(feat2d, idx_flat):
    n, d = idx_flat.shape[0], feat2d.shape[1]
    info = plsc.get_sparse_core_info()
    nc = info.num_cores
    nw = nc * info.num_subcores
    per_w = n // nw
    ch = 256
    n_iter = per_w // ch
    mesh = plsc.VectorSubcoreMesh(core_axis_name="c", subcore_axis_name="s")

    @functools.partial(
        pl.kernel,
        mesh=mesh,
        out_type=jax.ShapeDtypeStruct((n, d), jnp.float32),
        scratch_types=[
            pltpu.VMEM((ch,), jnp.int32),
            pltpu.VMEM((ch, d), jnp.float32),
            pltpu.SemaphoreType.DMA,
        ],
    )
    def kk(feat_hbm, idx_hbm, outf_hbm, idx_v, rows_v, sem):
        wid = lax.axis_index("s") * nc + lax.axis_index("c")
        base = wid * per_w

        def body(i, _):
            off = pl.multiple_of(base + i * ch, ch)
            pltpu.sync_copy(idx_hbm.at[pl.ds(off, ch)], idx_v)
            pltpu.async_copy(feat_hbm.at[idx_v], rows_v, sem).wait()
            pltpu.sync_copy(rows_v, outf_hbm.at[pl.ds(off, ch)])
            return 0

        lax.fori_loop(0, n_iter, body, 0)

    return kk(feat2d, idx_flat)


def _assemble_body(g_ref, x_ref, out_ref):
    tq = x_ref.shape[2]
    k = _NSAMPLE
    c = g_ref.shape[2]
    g = g_ref[0]
    x = x_ref[0]
    feat = jnp.swapaxes(g, 0, 1)
    feat = feat.reshape(c, tq, k)
    out_ref[0] = jnp.concatenate([x, feat], axis=0)


def _assemble(gfeat, gxyz, query_xyz, tq=64):
    B, nq, _ = query_xyz.shape
    k = _NSAMPLE
    c = gfeat.shape[1]
    g3 = gfeat.reshape(B, nq * k, c)
    nt = nq // tq
    return pl.pallas_call(
        _assemble_body,
        grid=(B, nt),
        in_specs=[
            pl.BlockSpec((1, tq * k, c), lambda b, t: (b, t, 0)),
            pl.BlockSpec((1, 3, tq, k), lambda b, t: (b, 0, t, 0)),
        ],
        out_specs=pl.BlockSpec((1, 3 + c, tq, k), lambda b, t: (b, 0, t, 0)),
        out_shape=jax.ShapeDtypeStruct((B, 3 + c, nq, k), jnp.float32),
        compiler_params=pltpu.CompilerParams(
            dimension_semantics=("parallel", "parallel")),
    )(g3, gxyz)


def kernel(query_xyz, support_xyz, query_mask, support_mask, features):
    B, nq, _ = query_xyz.shape
    ns = support_xyz.shape[1]
    C = features.shape[1]
    k = _NSAMPLE

    tq = 256
    support_t = jnp.swapaxes(support_xyz, 1, 2)
    smaskf = support_mask.astype(jnp.float32).reshape(B, 1, ns)
    qmaskf = query_mask.astype(jnp.float32).reshape(B * (nq // tq), 1, tq)

    idx, validf, gxyz = _ball_query(query_xyz, support_t, smaskf, qmaskf,
                                    tq=tq)

    feat2d = jnp.swapaxes(features, 1, 2).reshape(B * ns, C)
    gfeat = _sc_gather(feat2d, idx.reshape(B * nq * k))
    new_features = _assemble(gfeat, gxyz, query_xyz, tq=64)
    idx_mask = validf > 0.5
    return (new_features, idx_mask)
